# baseline (speedup 1.0000x reference)
import functools

import jax
import jax.numpy as jnp
from jax import lax
from jax.experimental import pallas as pl
from jax.experimental.pallas import tpu as pltpu
from jax.experimental.pallas import tpu_sc as plsc

B = 16384
D = 128
H = 16
NC = 2
NS = 16
NW = NC * NS
CHUNK = 128
NSPLIT = 2
BH = B // NSPLIT


def _make_gather_body(nrows, nbuf):
    bpw = nrows // NW
    nch = bpw // CHUNK
    nitems = 2 * nch

    def body(uidx, pidx, utab, ptab, u_out, p_out, idx_u, idx_p, *bufsem):
        rows = bufsem[:nbuf]
        gsem = bufsem[nbuf:2 * nbuf]
        ssem = bufsem[2 * nbuf:]
        wid = lax.axis_index("s") * NC + lax.axis_index("c")
        base = wid * bpw
        pltpu.sync_copy(uidx.at[pl.ds(nch * wid, nch)], idx_u)
        pltpu.sync_copy(pidx.at[pl.ds(nch * wid, nch)], idx_p)

        items = [(utab, idx_u, u_out, j) for j in range(nch)] + [
            (ptab, idx_p, p_out, j) for j in range(nch)
        ]

        def start_gather(i, b):
            tab, idx, _, j = items[i]
            pltpu.async_copy(tab.at[idx.at[j]], rows[b], gsem[b])

        def wait_gather(i, b):
            tab, idx, _, j = items[i]
            pltpu.make_async_copy(tab.at[idx.at[j]], rows[b], gsem[b]).wait()

        def wait_scatter(i, b):
            _, _, out, j = items[i]
            pltpu.make_async_copy(
                rows[b], out.at[pl.ds(base + j * CHUNK, CHUNK)], ssem[b]
            ).wait()

        for i in range(min(nbuf, nitems)):
            start_gather(i, i)
        pending = {}
        for i in range(nitems):
            b = i % nbuf
            wait_gather(i, b)
            _, _, out, j = items[i]
            pltpu.async_copy(
                rows[b], out.at[pl.ds(base + j * CHUNK, CHUNK)], ssem[b]
            )
            pending[b] = i
            if i + nbuf < nitems:
                wait_scatter(pending.pop(b), b)
                start_gather(i + nbuf, b)
        for b, i in pending.items():
            wait_scatter(i, b)

    return body, nch


@functools.cache
def _gather(nrows, nbuf):
    body, _ = _make_gather_body(nrows, nbuf)
    return functools.partial(
        pl.kernel,
        mesh=plsc.VectorSubcoreMesh(core_axis_name="c", subcore_axis_name="s"),
        out_type=[
            jax.ShapeDtypeStruct((nrows, D), jnp.float32),
            jax.ShapeDtypeStruct((nrows, D), jnp.float32),
        ],
        scratch_types=(
            [pltpu.VMEM((nrows // NW // CHUNK, CHUNK), jnp.int32)] * 2
            + [pltpu.VMEM((CHUNK, D), jnp.float32)] * nbuf
            + [pltpu.SemaphoreType.DMA] * (2 * nbuf)
        ),
    )(body)


BM = 4096


def _mlp_body(u_ref, p_ref, w1_ref, b1_ref, w2_ref, b2_ref, o_ref):
    h = jnp.dot(u_ref[...], w1_ref[:D, :], preferred_element_type=jnp.float32)
    h = h + jnp.dot(p_ref[...], w1_ref[D:, :], preferred_element_type=jnp.float32)
    h = jnp.maximum(h + b1_ref[...], 0.0)
    res = jnp.dot(h, w2_ref[...], preferred_element_type=jnp.float32) + b2_ref[...]
    o_ref[...] = res.reshape(BM)


def _mlp(u_rows, p_rows, W1, b1, W2, b2):
    n = u_rows.shape[0]
    return pl.pallas_call(
        _mlp_body,
        grid=(n // BM,),
        in_specs=[
            pl.BlockSpec((BM, D), lambda i: (i, 0)),
            pl.BlockSpec((BM, D), lambda i: (i, 0)),
            pl.BlockSpec((2 * D, H), lambda i: (0, 0)),
            pl.BlockSpec((1, H), lambda i: (0, 0)),
            pl.BlockSpec((H, 1), lambda i: (0, 0)),
            pl.BlockSpec((1, 1), lambda i: (0, 0)),
        ],
        out_specs=pl.BlockSpec((BM,), lambda i: (i,)),
        out_shape=jax.ShapeDtypeStruct((n,), jnp.float32),
    )(u_rows, p_rows, W1, b1, W2, b2)


def kernel(inputs, user_table, prod_table, W1, b1, W2, b2):
    uidx = inputs[:, 0].astype(jnp.int32).reshape(B // CHUNK, CHUNK)
    pidx = inputs[:, 1].astype(jnp.int32).reshape(B // CHUNK, CHUNK)
    b1r = b1.reshape(1, H)
    b2r = b2.reshape(1, 1)
    rh = BH // CHUNK
    gather = _gather(BH, 4)
    halves = []
    for s in range(NSPLIT):
        u_rows, p_rows = gather(
            uidx[s * rh:(s + 1) * rh], pidx[s * rh:(s + 1) * rh],
            user_table, prod_table,
        )
        halves.append((u_rows, p_rows))
    outs = [_mlp(u, p, W1, b1r, W2, b2r) for u, p in halves]
    return jnp.concatenate(outs).reshape(B, 1)


# speedup vs baseline: 1.0709x; 1.0709x over previous
import functools

import jax
import jax.numpy as jnp
from jax import lax
from jax.experimental import pallas as pl
from jax.experimental.pallas import tpu as pltpu
from jax.experimental.pallas import tpu_sc as plsc

B = 16384
D = 128
H = 16
NC = 2
NS = 16
NW = NC * NS
BPW = B // NW
CHUNK = 128
NCH = BPW // CHUNK
NBUF = 7
NITEMS = 2 * NCH
L = 16


def _gather_body(uidx, pidx, utab, ptab, u_out, p_out, idx_u, idx_p,
                 *bufsem):
    rows = bufsem[:NBUF]
    gsem = bufsem[NBUF:2 * NBUF]
    ssem = bufsem[2 * NBUF:]
    wid = lax.axis_index("s") * NC + lax.axis_index("c")
    base = wid * BPW
    pltpu.sync_copy(uidx.at[pl.ds(NCH * wid, NCH)], idx_u)
    pltpu.sync_copy(pidx.at[pl.ds(NCH * wid, NCH)], idx_p)

    items = [(utab, idx_u, u_out, j) for j in range(NCH)] + [
        (ptab, idx_p, p_out, j) for j in range(NCH)
    ]

    def start_gather(i, b):
        tab, idx, _, j = items[i]
        pltpu.async_copy(tab.at[idx.at[j]], rows[b], gsem[b])

    def wait_gather(i, b):
        tab, idx, _, j = items[i]
        pltpu.make_async_copy(tab.at[idx.at[j]], rows[b], gsem[b]).wait()

    def wait_scatter(i, b):
        _, _, out, j = items[i]
        pltpu.make_async_copy(
            rows[b], out.at[pl.ds(base + j * CHUNK, CHUNK)], ssem[b]
        ).wait()

    for i in range(min(NBUF, NITEMS)):
        start_gather(i, i)
    pending = {}
    for i in range(NITEMS):
        b = i % NBUF
        wait_gather(i, b)
        _, _, out, j = items[i]
        pltpu.async_copy(
            rows[b], out.at[pl.ds(base + j * CHUNK, CHUNK)], ssem[b]
        )
        pending[b] = i
        if i + NBUF < NITEMS:
            wait_scatter(pending.pop(b), b)
            start_gather(i + NBUF, b)
    for b, i in pending.items():
        wait_scatter(i, b)


@functools.cache
def _gather():
    return functools.partial(
        pl.kernel,
        mesh=plsc.VectorSubcoreMesh(core_axis_name="c", subcore_axis_name="s"),
        out_type=[
            jax.ShapeDtypeStruct((B, D), jnp.float32),
            jax.ShapeDtypeStruct((B, D), jnp.float32),
        ],
        scratch_types=(
            [pltpu.VMEM((NCH, CHUNK), jnp.int32)] * 2
            + [pltpu.VMEM((CHUNK, D), jnp.float32)] * NBUF
            + [pltpu.SemaphoreType.DMA] * (2 * NBUF)
        ),
    )(_gather_body)


BM = 4096


def _mlp_body(u_ref, p_ref, w1_ref, b1_ref, w2_ref, b2_ref, o_ref):
    h = jnp.dot(u_ref[...], w1_ref[:D, :], preferred_element_type=jnp.float32)
    h = h + jnp.dot(p_ref[...], w1_ref[D:, :], preferred_element_type=jnp.float32)
    h = jnp.maximum(h + b1_ref[...], 0.0)
    res = jnp.dot(h, w2_ref[...], preferred_element_type=jnp.float32) + b2_ref[...]
    o_ref[...] = res.reshape(BM)


def _mlp(u_rows, p_rows, W1, b1, W2, b2):
    return pl.pallas_call(
        _mlp_body,
        grid=(B // BM,),
        in_specs=[
            pl.BlockSpec((BM, D), lambda i: (i, 0)),
            pl.BlockSpec((BM, D), lambda i: (i, 0)),
            pl.BlockSpec((2 * D, H), lambda i: (0, 0)),
            pl.BlockSpec((1, H), lambda i: (0, 0)),
            pl.BlockSpec((H, 1), lambda i: (0, 0)),
            pl.BlockSpec((1, 1), lambda i: (0, 0)),
        ],
        out_specs=pl.BlockSpec((BM,), lambda i: (i,)),
        out_shape=jax.ShapeDtypeStruct((B,), jnp.float32),
    )(u_rows, p_rows, W1, b1, W2, b2)


def kernel(inputs, user_table, prod_table, W1, b1, W2, b2):
    uidx = inputs[:, 0].astype(jnp.int32).reshape(B // CHUNK, CHUNK)
    pidx = inputs[:, 1].astype(jnp.int32).reshape(B // CHUNK, CHUNK)
    u_rows, p_rows = _gather()(uidx, pidx, user_table, prod_table)
    out = _mlp(u_rows, p_rows, W1, b1.reshape(1, H), W2, b2.reshape(1, 1))
    return out.reshape(B, 1)
